# Initial kernel scaffold; baseline (speedup 1.0000x reference)
import jax
import jax.numpy as jnp
from jax.experimental import pallas as pl


def kernel(boxes, scores):
    raise NotImplementedError("write your pallas kernel here")


# speedup vs baseline: 23.2550x; 23.2550x over previous
import jax
import jax.numpy as jnp
from jax.experimental import pallas as pl
from jax.experimental.pallas import tpu as pltpu

_N_PRE = 2000
_PAD = 2048
_N_POST = 1000
_NMS_T = 0.7
_MIN = 1e-3
_IMG = 1024.0
_BIG = 1e9


def _rpn_nms_body(bxt_ref, sc_ref, out_ref, cl_ref, alive_ref, cnt_ref):
    x1 = jnp.clip(bxt_ref[0:1, :], 0.0, _IMG)
    y1 = jnp.clip(bxt_ref[1:2, :], 0.0, _IMG)
    x2 = jnp.clip(bxt_ref[2:3, :], 0.0, _IMG)
    y2 = jnp.clip(bxt_ref[3:4, :], 0.0, _IMG)
    probs = sc_ref[0:1, :]
    ws = x2 - x1
    hs = y2 - y1
    valid = (ws >= _MIN) & (hs >= _MIN) & (probs > 0.0)
    area = (x2 - x1) * (y2 - y1)
    cl_ref[0:1, :] = x1
    cl_ref[1:2, :] = y1
    cl_ref[2:3, :] = x2
    cl_ref[3:4, :] = y2
    cl_ref[4:5, :] = area
    alive_ref[...] = jnp.where(valid, probs, -_BIG)
    out_ref[...] = jnp.zeros(out_ref.shape, jnp.float32)
    cnt_ref[0] = 0

    lane = jax.lax.broadcasted_iota(jnp.int32, (1, _PAD), 1)
    cols8 = jax.lax.broadcasted_iota(jnp.int32, (1, 8), 1)

    def step(i, carry):
        alive = alive_ref[...]
        onehot = lane == i
        s_i = jnp.sum(jnp.where(onehot, alive, 0.0))
        keep = (s_i > (-_BIG / 2.0)) & (cnt_ref[0] < _N_POST)

        @pl.when(keep)
        def _():
            xs1 = cl_ref[0:1, :]
            ys1 = cl_ref[1:2, :]
            xs2 = cl_ref[2:3, :]
            ys2 = cl_ref[3:4, :]
            ar = cl_ref[4:5, :]
            xi1 = jnp.sum(jnp.where(onehot, xs1, 0.0))
            yi1 = jnp.sum(jnp.where(onehot, ys1, 0.0))
            xi2 = jnp.sum(jnp.where(onehot, xs2, 0.0))
            yi2 = jnp.sum(jnp.where(onehot, ys2, 0.0))
            ai = (xi2 - xi1) * (yi2 - yi1)
            xx1 = jnp.maximum(xi1, xs1)
            yy1 = jnp.maximum(yi1, ys1)
            xx2 = jnp.minimum(xi2, xs2)
            yy2 = jnp.minimum(yi2, ys2)
            inter = jnp.clip(xx2 - xx1, 0.0, None) * jnp.clip(yy2 - yy1, 0.0, None)
            iou = inter / (ai + ar - inter + 1e-9)
            supp = iou > _NMS_T
            alive_ref[...] = jnp.where(supp, -_BIG, alive_ref[...])
            c = cnt_ref[0]
            row = (jnp.where(cols8 == 0, xi1, 0.0)
                   + jnp.where(cols8 == 1, yi1, 0.0)
                   + jnp.where(cols8 == 2, xi2, 0.0)
                   + jnp.where(cols8 == 3, yi2, 0.0)
                   + jnp.where(cols8 == 4, s_i, 0.0))
            out_ref[pl.ds(c, 1), :] = row
            cnt_ref[0] = c + 1

        return carry

    jax.lax.fori_loop(0, _N_PRE, step, 0)


def kernel(boxes, scores):
    probs = jax.nn.sigmoid(scores)
    top_probs, idx = jax.lax.top_k(probs, _N_PRE)
    top_boxes = jnp.take(boxes, idx, axis=0)

    bxt = jnp.zeros((8, _PAD), jnp.float32).at[0:4, :_N_PRE].set(top_boxes.T)
    sc = jnp.full((1, _PAD), -1.0, jnp.float32).at[0, :_N_PRE].set(top_probs)

    out = pl.pallas_call(
        _rpn_nms_body,
        out_shape=jax.ShapeDtypeStruct((1024, 8), jnp.float32),
        scratch_shapes=[
            pltpu.VMEM((8, _PAD), jnp.float32),
            pltpu.VMEM((1, _PAD), jnp.float32),
            pltpu.SMEM((1,), jnp.int32),
        ],
    )(bxt, sc)
    return out[:_N_POST, :5]


# speedup vs baseline: 37.0847x; 1.5947x over previous
import jax
import jax.numpy as jnp
from jax.experimental import pallas as pl
from jax.experimental.pallas import tpu as pltpu

_N_PRE = 2000
_PAD = 2048
_N_POST = 1000
_NMS_T = 0.7
_MIN = 1e-3
_IMG = 1024.0
_BIG = 1e9


def _rpn_nms_body(bxt_ref, bxn_ref, sc_ref, out_ref, cl_ref, alive_ref):
    x1 = jnp.clip(bxt_ref[0:1, :], 0.0, _IMG)
    y1 = jnp.clip(bxt_ref[1:2, :], 0.0, _IMG)
    x2 = jnp.clip(bxt_ref[2:3, :], 0.0, _IMG)
    y2 = jnp.clip(bxt_ref[3:4, :], 0.0, _IMG)
    probs = sc_ref[0:1, :]
    ws = x2 - x1
    hs = y2 - y1
    valid = (ws >= _MIN) & (hs >= _MIN) & (probs > 0.0)
    area = (x2 - x1) * (y2 - y1)
    cl_ref[0:1, :] = x1
    cl_ref[1:2, :] = y1
    cl_ref[2:3, :] = x2
    cl_ref[3:4, :] = y2
    cl_ref[4:5, :] = area
    alive_ref[...] = jnp.where(valid, probs, -_BIG)
    out_ref[...] = jnp.zeros(out_ref.shape, jnp.float32)

    cols8 = jax.lax.broadcasted_iota(jnp.int32, (1, 8), 1)
    lane = jax.lax.broadcasted_iota(jnp.int32, (1, _PAD), 1)

    def cond(state):
        i, c = state
        return (i < _N_PRE) & (c < _N_POST)

    def step(state):
        i, c = state
        s_i = jnp.sum(jnp.where(lane == i, alive_ref[...], 0.0))
        keep = s_i > (-_BIG / 2.0)

        @pl.when(keep)
        def _():
            xs1 = cl_ref[0:1, :]
            ys1 = cl_ref[1:2, :]
            xs2 = cl_ref[2:3, :]
            ys2 = cl_ref[3:4, :]
            ar = cl_ref[4:5, :]
            rowv = jnp.clip(bxn_ref[pl.ds(i, 1), :], 0.0, _IMG)
            xi1 = rowv[0, 0]
            yi1 = rowv[0, 1]
            xi2 = rowv[0, 2]
            yi2 = rowv[0, 3]
            ai = (xi2 - xi1) * (yi2 - yi1)
            xx1 = jnp.maximum(xi1, xs1)
            yy1 = jnp.maximum(yi1, ys1)
            xx2 = jnp.minimum(xi2, xs2)
            yy2 = jnp.minimum(yi2, ys2)
            inter = jnp.clip(xx2 - xx1, 0.0, None) * jnp.clip(yy2 - yy1, 0.0, None)
            iou = inter / (ai + ar - inter + 1e-9)
            supp = iou > _NMS_T
            alive_ref[...] = jnp.where(supp, -_BIG, alive_ref[...])
            row = (jnp.where(cols8 == 0, xi1, 0.0)
                   + jnp.where(cols8 == 1, yi1, 0.0)
                   + jnp.where(cols8 == 2, xi2, 0.0)
                   + jnp.where(cols8 == 3, yi2, 0.0)
                   + jnp.where(cols8 == 4, s_i, 0.0))
            out_ref[pl.ds(c, 1), :] = row

        return (i + 1, jnp.where(keep, c + 1, c))

    jax.lax.while_loop(cond, step, (0, 0))


def kernel(boxes, scores):
    probs = jax.nn.sigmoid(scores)
    top_probs, idx = jax.lax.top_k(probs, _N_PRE)
    top_boxes = jnp.take(boxes, idx, axis=0)

    bxt = jnp.zeros((8, _PAD), jnp.float32).at[0:4, :_N_PRE].set(top_boxes.T)
    bxn = jnp.zeros((_PAD, 8), jnp.float32).at[:_N_PRE, 0:4].set(top_boxes)
    sc = jnp.full((1, _PAD), -1.0, jnp.float32).at[0, :_N_PRE].set(top_probs)

    out = pl.pallas_call(
        _rpn_nms_body,
        out_shape=jax.ShapeDtypeStruct((1024, 8), jnp.float32),
        scratch_shapes=[
            pltpu.VMEM((8, _PAD), jnp.float32),
            pltpu.VMEM((1, _PAD), jnp.float32),
        ],
    )(bxt, bxn, sc)
    return out[:_N_POST, :5]
